# Initial kernel scaffold; baseline (speedup 1.0000x reference)
import jax
import jax.numpy as jnp
from jax.experimental import pallas as pl


def kernel(acts, W_enc, W_dec, b_enc, b_dec):
    raise NotImplementedError("write your pallas kernel here")


# speedup vs baseline: 2.2069x; 2.2069x over previous
import functools

import jax
import jax.numpy as jnp
from jax import lax
from jax.experimental import pallas as pl

D_IN = 2048
D_SAE = 65536
B_TOK = 64
K_TOP = 64

BN_ENC = 1024
BK_DEC = 1024
R_SLAB = 8


def _encode_body(acts_ref, w_ref, benc_ref, bdec_ref, out_ref):
    x = acts_ref[...] - bdec_ref[...]
    pre = jnp.dot(x, w_ref[...], preferred_element_type=jnp.float32)
    out_ref[...] = jnp.maximum(pre + benc_ref[...], 0.0)


def _thresh_body(a_ref, out_ref):
    ai = lax.bitcast_convert_type(a_ref[...], jnp.int32)

    def step(it, lo):
        j = 30 - it
        mid = lo + jnp.left_shift(jnp.int32(1), j)
        cnt = jnp.sum((ai >= mid).astype(jnp.int32), axis=1, keepdims=True)
        return jnp.where(cnt >= K_TOP, mid, lo)

    lo = lax.fori_loop(0, 31, step, jnp.zeros((R_SLAB, 1), jnp.int32))
    t = lax.bitcast_convert_type(lo, jnp.float32)
    out_ref[...] = jnp.broadcast_to(t, (R_SLAB, 128))


def _decode_body(a_ref, t_ref, w_ref, bdec_ref, out_ref):
    i = pl.program_id(0)
    t = t_ref[...][:, 0:1]
    a = a_ref[...]
    s = jnp.where(a >= t, a, 0.0)
    contrib = jnp.dot(s, w_ref[...], preferred_element_type=jnp.float32)

    @pl.when(i == 0)
    def _():
        out_ref[...] = jnp.broadcast_to(bdec_ref[...], out_ref.shape)

    out_ref[...] += contrib


def kernel(acts, W_enc, W_dec, b_enc, b_dec):
    b_enc2 = b_enc.reshape(1, D_SAE)
    b_dec2 = b_dec.reshape(1, D_IN)

    a = pl.pallas_call(
        _encode_body,
        grid=(D_SAE // BN_ENC,),
        in_specs=[
            pl.BlockSpec((B_TOK, D_IN), lambda i: (0, 0)),
            pl.BlockSpec((D_IN, BN_ENC), lambda i: (0, i)),
            pl.BlockSpec((1, BN_ENC), lambda i: (0, i)),
            pl.BlockSpec((1, D_IN), lambda i: (0, 0)),
        ],
        out_specs=pl.BlockSpec((B_TOK, BN_ENC), lambda i: (0, i)),
        out_shape=jax.ShapeDtypeStruct((B_TOK, D_SAE), jnp.float32),
    )(acts, W_enc, b_enc2, b_dec2)

    thresh = pl.pallas_call(
        _thresh_body,
        grid=(B_TOK // R_SLAB,),
        in_specs=[pl.BlockSpec((R_SLAB, D_SAE), lambda i: (i, 0))],
        out_specs=pl.BlockSpec((R_SLAB, 128), lambda i: (i, 0)),
        out_shape=jax.ShapeDtypeStruct((B_TOK, 128), jnp.float32),
    )(a)

    recon = pl.pallas_call(
        _decode_body,
        grid=(D_SAE // BK_DEC,),
        in_specs=[
            pl.BlockSpec((B_TOK, BK_DEC), lambda i: (0, i)),
            pl.BlockSpec((B_TOK, 128), lambda i: (0, 0)),
            pl.BlockSpec((BK_DEC, D_IN), lambda i: (i, 0)),
            pl.BlockSpec((1, D_IN), lambda i: (0, 0)),
        ],
        out_specs=pl.BlockSpec((B_TOK, D_IN), lambda i: (0, 0)),
        out_shape=jax.ShapeDtypeStruct((B_TOK, D_IN), jnp.float32),
    )(a, thresh, W_dec, b_dec2)

    return recon
